# baseline (speedup 1.0000x reference)
import jax
import jax.numpy as jnp
from jax.experimental import pallas as pl
from jax.experimental.pallas import tpu as pltpu

_N = 1000000
_K = 16384
_TAU_MEANS = 4.0
_EPS_ALPHA = 0.005
_PHI = 1.6
_INTERVAL_DENSIFY = 100.0
_MAX_WH = 1600.0
_SCALE_THRESH = 0.01

_BLK = 131072


def _score_kernel(mga_ref, op_ref, st_ref, c_ref, s_ref):
    g = mga_ref[...] / _INTERVAL_DENSIFY / 2.0 * _MAX_WH
    grad_mask = g >= _TAU_MEANS
    prune_mask = jax.nn.sigmoid(op_ref[...]) > _EPS_ALPHA
    scale_max = jnp.max(jnp.exp(st_ref[...]), axis=0)
    base = prune_mask & grad_mask
    clone = (scale_max < _SCALE_THRESH) & base
    split = (scale_max > _SCALE_THRESH) & base
    neg = jnp.float32(-jnp.inf)
    c_ref[...] = jnp.where(clone, g, neg)
    s_ref[...] = jnp.where(split, g, neg)


def _gather_kernel(idx_ref, feats_ref, valid_ref, out_ref):
    i = pl.program_id(0)
    row = feats_ref[...]
    col = jax.lax.broadcasted_iota(jnp.int32, row.shape, 2)
    split_row = jnp.log(jnp.exp(row) / _PHI)
    sel = (col >= 6) & (col < 9) & (i >= _K)
    row = jnp.where(sel, split_row, row)
    v = valid_ref[0, 0, 0] > 0.0
    out_ref[...] = jnp.where(v, row, jnp.zeros_like(row))


def kernel(means, colors_dc, scales, quats, opacities, means_grad_accum):
    mga = means_grad_accum
    op0 = opacities[:, 0]
    st = scales.T

    grid = pl.cdiv(_N, _BLK)
    c_score, s_score = pl.pallas_call(
        _score_kernel,
        grid=(grid,),
        in_specs=[
            pl.BlockSpec((_BLK,), lambda i: (i,)),
            pl.BlockSpec((_BLK,), lambda i: (i,)),
            pl.BlockSpec((3, _BLK), lambda i: (0, i)),
        ],
        out_specs=[
            pl.BlockSpec((_BLK,), lambda i: (i,)),
            pl.BlockSpec((_BLK,), lambda i: (i,)),
        ],
        out_shape=[
            jax.ShapeDtypeStruct((_N,), jnp.float32),
            jax.ShapeDtypeStruct((_N,), jnp.float32),
        ],
    )(mga, op0, st)

    c_vals, c_idx = jax.lax.top_k(c_score, _K)
    s_vals, s_idx = jax.lax.top_k(s_score, _K)

    feats = jnp.concatenate([means, colors_dc, scales, quats, opacities], axis=-1)
    feats3 = feats[:, None, :]
    idx = jnp.concatenate([c_idx, s_idx]).astype(jnp.int32)
    valid = jnp.concatenate(
        [jnp.isfinite(c_vals), jnp.isfinite(s_vals)]
    ).astype(jnp.float32)[:, None, None]

    out = pl.pallas_call(
        _gather_kernel,
        grid_spec=pltpu.PrefetchScalarGridSpec(
            num_scalar_prefetch=1,
            grid=(2 * _K,),
            in_specs=[
                pl.BlockSpec((1, 1, 14), lambda i, idx_ref: (idx_ref[i], 0, 0)),
                pl.BlockSpec((1, 1, 1), lambda i, idx_ref: (i, 0, 0)),
            ],
            out_specs=pl.BlockSpec((1, 1, 14), lambda i, idx_ref: (i, 0, 0)),
        ),
        out_shape=jax.ShapeDtypeStruct((2 * _K, 1, 14), jnp.float32),
    )(idx, feats3, valid)
    return out[:, 0, :]


# speedup vs baseline: 4.3572x; 4.3572x over previous
import jax
import jax.numpy as jnp
from jax.experimental import pallas as pl
from jax.experimental.pallas import tpu as pltpu

_N = 1000000
_K = 16384
_TAU_MEANS = 4.0
_EPS_ALPHA = 0.005
_PHI = 1.6
_INTERVAL_DENSIFY = 100.0
_MAX_WH = 1600.0
_SCALE_THRESH = 0.01

_BLK = 131072


def _score_kernel(mga_ref, op_ref, st_ref, c_ref, s_ref):
    g = mga_ref[...] / _INTERVAL_DENSIFY / 2.0 * _MAX_WH
    grad_mask = g >= _TAU_MEANS
    prune_mask = jax.nn.sigmoid(op_ref[...]) > _EPS_ALPHA
    scale_max = jnp.max(jnp.exp(st_ref[...]), axis=0)
    base = prune_mask & grad_mask
    clone = (scale_max < _SCALE_THRESH) & base
    split = (scale_max > _SCALE_THRESH) & base
    neg = jnp.float32(-jnp.inf)
    c_ref[...] = jnp.where(clone, g, neg)
    s_ref[...] = jnp.where(split, g, neg)


_R = 128


def _gather_kernel(idx_ref, valid_ref, feats_ref, out_ref, scratch, sem):
    b = pl.program_id(0)

    def issue(r, carry):
        gidx = idx_ref[b * _R + r]
        pltpu.make_async_copy(
            feats_ref.at[pl.ds(gidx, 1)], scratch.at[pl.ds(r, 1)], sem
        ).start()
        return carry

    jax.lax.fori_loop(0, _R, issue, 0)

    def wait(r, carry):
        pltpu.make_async_copy(
            feats_ref.at[pl.ds(0, 1)], scratch.at[pl.ds(0, 1)], sem
        ).wait()
        return carry

    jax.lax.fori_loop(0, _R, wait, 0)

    rows = scratch[...]
    col = jax.lax.broadcasted_iota(jnp.int32, rows.shape, 1)
    split_rows = jnp.log(jnp.exp(rows) / _PHI)
    sel = (col >= 6) & (col < 9) & (b * _R >= _K)
    rows = jnp.where(sel, split_rows, rows)
    v = valid_ref[...] > 0.0
    out_ref[...] = jnp.where(v, rows, jnp.zeros_like(rows))


def kernel(means, colors_dc, scales, quats, opacities, means_grad_accum):
    mga = means_grad_accum
    op0 = opacities[:, 0]
    st = scales.T

    grid = pl.cdiv(_N, _BLK)
    c_score, s_score = pl.pallas_call(
        _score_kernel,
        grid=(grid,),
        in_specs=[
            pl.BlockSpec((_BLK,), lambda i: (i,)),
            pl.BlockSpec((_BLK,), lambda i: (i,)),
            pl.BlockSpec((3, _BLK), lambda i: (0, i)),
        ],
        out_specs=[
            pl.BlockSpec((_BLK,), lambda i: (i,)),
            pl.BlockSpec((_BLK,), lambda i: (i,)),
        ],
        out_shape=[
            jax.ShapeDtypeStruct((_N,), jnp.float32),
            jax.ShapeDtypeStruct((_N,), jnp.float32),
        ],
    )(mga, op0, st)

    c_vals, c_idx = jax.lax.top_k(c_score, _K)
    s_vals, s_idx = jax.lax.top_k(s_score, _K)

    feats = jnp.concatenate([means, colors_dc, scales, quats, opacities], axis=-1)
    idx = jnp.concatenate([c_idx, s_idx]).astype(jnp.int32)
    valid = jnp.concatenate(
        [jnp.isfinite(c_vals), jnp.isfinite(s_vals)]
    ).astype(jnp.float32)[:, None]

    out = pl.pallas_call(
        _gather_kernel,
        grid_spec=pltpu.PrefetchScalarGridSpec(
            num_scalar_prefetch=1,
            grid=(2 * _K // _R,),
            in_specs=[
                pl.BlockSpec((_R, 1), lambda b, idx_ref: (b, 0)),
                pl.BlockSpec(memory_space=pl.ANY),
            ],
            out_specs=pl.BlockSpec((_R, 14), lambda b, idx_ref: (b, 0)),
            scratch_shapes=[
                pltpu.VMEM((_R, 14), jnp.float32),
                pltpu.SemaphoreType.DMA,
            ],
        ),
        out_shape=jax.ShapeDtypeStruct((2 * _K, 14), jnp.float32),
    )(idx, valid, feats)
    return out
